# baseline (speedup 1.0000x reference)
import functools

import jax
import jax.numpy as jnp
from jax import lax
from jax.experimental import pallas as pl
from jax.experimental.pallas import tpu as pltpu
from jax.experimental.pallas import tpu_sc as plsc

_N = 10000
_G = 8
_NC = 2
_NS = 16
_NW = _NC * _NS
_CH = 320
_NBLK = _N // 16
_NPAD = _N + 16
_MARGIN = 0.1
_NEG = -1e30


def _tree_sum(vs):
    while len(vs) > 1:
        vs = [vs[i] + vs[i + 1] for i in range(0, len(vs) - 1, 2)] + (
            [vs[-1]] if len(vs) % 2 else [])
    return vs[0]


def _sc_body(r_hbm, lab_hbm, bat_hbm, part_hbm,
             r_v, bs_v, lab_v, bat_v, out_v):
    wid = lax.axis_index("s") * _NC + lax.axis_index("c")
    iot = lax.iota(jnp.int32, 16)
    zero16 = jnp.zeros((16,), jnp.float32)

    pltpu.sync_copy(r_hbm, r_v.at[pl.ds(0, _N)])
    pltpu.sync_copy(lab_hbm, lab_v.at[pl.ds(0, _N)])
    pltpu.sync_copy(bat_hbm, bat_v.at[pl.ds(0, _N)])

    def bs_step(k, _):
        sl = pl.ds(k * 16, 16)
        bs_v[sl] = jnp.where(lab_v[sl] == 0, r_v[sl], _NEG)
        return 0
    lax.fori_loop(0, _NBLK, bs_step, 0)

    starts = [jnp.int32(0)]
    for g in range(1, _G):
        def bstep(_, lohi, g=g):
            lo, hi = lohi
            mid = (lo + hi) // 2
            pred = bat_v[pl.ds(mid, 16)][0] < g
            return (jnp.where(pred, mid + 1, lo), jnp.where(pred, hi, mid))
        lo, _hi = lax.fori_loop(0, 14, bstep, (jnp.int32(0), jnp.int32(_N)))
        starts.append(lo)
    starts.append(jnp.int32(_N))

    base_i = wid * _CH

    num_vecs = []
    gold_vecs = []
    for g in range(_G):
        st = starts[g]
        en = starts[g + 1]
        a0 = st // 16
        b0 = en // 16
        jvA = a0 * 16 + iot
        vA = bs_v[pl.ds(a0 * 16, 16)]
        vA = jnp.where(jvA >= st, vA, _NEG)
        vA = jnp.where(jvA < en, vA, _NEG)
        jvB = b0 * 16 + iot
        vB = bs_v[pl.ds(b0 * 16, 16)]
        vB = jnp.where(jvB >= (a0 + 1) * 16, vB, _NEG)
        vB = jnp.where(jvB < en, vB, _NEG)

        rlo = jnp.maximum(base_i, st)
        rhi = jnp.minimum(base_i + _CH, en)
        nrb = jnp.maximum(0, (rhi - rlo + 15) // 16)

        def rb_step(rb, c, rlo=rlo, rhi=rhi, a0=a0, b0=b0, vA=vA, vB=vB):
            nacc, gacc = c
            rbase = rlo + rb * 16
            rowjv = rbase + iot
            av = r_v[pl.ds(rbase, 16)]
            lv = lab_v[pl.ds(rbase, 16)]
            c0 = _MARGIN - av
            c0 = jnp.where(lv > 0, c0, _NEG)
            c0 = jnp.where(rowjv < rhi, c0, _NEG)
            c0b = [c0[k] + zero16 for k in range(16)]

            nin = jnp.maximum(0, b0 - (a0 + 1))
            jbase0 = (a0 + 1) * 16

            def jstep2(k, accs, jbase0=jbase0):
                b1 = bs_v[pl.ds(jbase0 + k * 32, 16)]
                b2 = bs_v[pl.ds(jbase0 + k * 32 + 16, 16)]
                accs = tuple(accs[k2] + jnp.maximum(b1 + c0b[k2], 0.0)
                             for k2 in range(16))
                return tuple(accs[k2] + jnp.maximum(b2 + c0b[k2], 0.0)
                             for k2 in range(16))

            def jtail(t, accs, jbase0=jbase0, nin=nin):
                b = bs_v[pl.ds(jbase0 + (nin - 1) * 16, 16)]
                return tuple(accs[k2] + jnp.maximum(b + c0b[k2], 0.0)
                             for k2 in range(16))

            accs = lax.fori_loop(0, nin // 2, jstep2, (zero16,) * 16)
            accs = lax.fori_loop(0, nin % 2, jtail, accs)
            for ev in (vA, vB):
                accs = tuple(accs[k] + jnp.maximum(ev + c0b[k], 0.0)
                             for k in range(16))
            nacc = nacc + _tree_sum(list(accs))
            rmask = jnp.where(rowjv < rhi, jnp.float32(1.0), jnp.float32(0.0))
            gacc = gacc + jnp.where(lv > 0, rmask, 0.0)
            return (nacc, gacc)

        nv, gv = lax.fori_loop(0, nrb, rb_step, (zero16, zero16))
        num_vecs.append(nv)
        gold_vecs.append(gv)

    for g in range(_G):
        out_v[pl.ds(g * 16, 16)] = num_vecs[g]
        out_v[pl.ds(128 + g * 16, 16)] = gold_vecs[g]
    pltpu.sync_copy(out_v, part_hbm.at[wid])


_sc_kernel = functools.partial(
    pl.kernel,
    out_type=jax.ShapeDtypeStruct((_NW, 256), jnp.float32),
    mesh=plsc.VectorSubcoreMesh(
        core_axis_name="c", subcore_axis_name="s",
        num_cores=_NC, num_subcores=_NS),
    scratch_types=[
        pltpu.VMEM((_NPAD,), jnp.float32),
        pltpu.VMEM((_NPAD,), jnp.float32),
        pltpu.VMEM((_NPAD,), jnp.int32),
        pltpu.VMEM((_NPAD,), jnp.int32),
        pltpu.VMEM((256,), jnp.float32),
    ],
)(_sc_body)


def _combine_body(part_ref, bat_ref, r_ref, o_ref, out_ref):
    p = part_ref[...]
    b = bat_ref[...]
    d = r_ref[...] - o_ref[...]
    asum = jnp.sum(d * d)
    rsum = jnp.sum(jnp.abs(d))
    rank_num = jnp.float32(0.0)
    rank_den = jnp.float32(0.0)
    for g in range(_G):
        num_g = jnp.sum(p[:, g * 16:(g + 1) * 16])
        gold_g = jnp.sum(p[:, 128 + g * 16:144 + g * 16])
        size_g = jnp.sum((b == g).astype(jnp.float32))
        cnt = gold_g * (size_g - gold_g)
        lg = jnp.where(cnt > 0, num_g / jnp.maximum(cnt, 1.0), 0.0)
        rank_num = rank_num + lg
        rank_den = rank_den + jnp.where(cnt > 0, 1.0, 0.0)
    rank = jnp.where(rank_den > 0, rank_num / jnp.maximum(rank_den, 1.0), 0.0)
    align = asum / _N
    reg = rsum / _N
    total = rank + 0.5 * align + 0.1 * reg
    ri = lax.broadcasted_iota(jnp.int32, (8, 128), 0)
    li = lax.broadcasted_iota(jnp.int32, (8, 128), 1)
    vals = (jnp.where(li == 0, total, 0.0) + jnp.where(li == 1, rank, 0.0)
            + jnp.where(li == 2, align, 0.0) + jnp.where(li == 3, reg, 0.0))
    out_ref[...] = jnp.where(ri == 0, vals, 0.0)


def _combine(part, bat_pad, r_pad, o_pad):
    return pl.pallas_call(
        _combine_body,
        out_shape=jax.ShapeDtypeStruct((8, 128), jnp.float32),
    )(part, bat_pad, r_pad, o_pad)


def kernel(refined_scores, original_scores, node_labels, batch):
    r = refined_scores.astype(jnp.float32)
    o = original_scores.astype(jnp.float32)
    lab = node_labels.astype(jnp.int32)
    bat = batch.astype(jnp.int32)
    part = _sc_kernel(r, lab, bat)
    pad = _NW * _CH - _N
    bat_pad = jnp.pad(bat, (0, pad), constant_values=_G).reshape(80, 128)
    r_pad = jnp.pad(r, (0, pad)).reshape(80, 128)
    o_pad = jnp.pad(o, (0, pad)).reshape(80, 128)
    res = _combine(part, bat_pad, r_pad, o_pad)
    return (res[0, 0], res[0, 1], res[0, 2], res[0, 3])


# speedup vs baseline: 1.0625x; 1.0625x over previous
import functools

import jax
import jax.numpy as jnp
from jax import lax
from jax.experimental import pallas as pl
from jax.experimental.pallas import tpu as pltpu
from jax.experimental.pallas import tpu_sc as plsc

_N = 10000
_G = 8
_NC = 2
_NS = 16
_NW = _NC * _NS
_CH = 320
_NBLK = _N // 16
_NPAD = _N + 16
_MARGIN = 0.1
_NEG = -1e30


def _tree_sum(vs):
    while len(vs) > 1:
        vs = [vs[i] + vs[i + 1] for i in range(0, len(vs) - 1, 2)] + (
            [vs[-1]] if len(vs) % 2 else [])
    return vs[0]


def _sc_body(r_hbm, lab_hbm, bat_hbm, part_hbm,
             r_v, bs_v, lab_v, bat_v, bnd_v, out_v):
    wid = lax.axis_index("s") * _NC + lax.axis_index("c")
    iot = lax.iota(jnp.int32, 16)
    zero16 = jnp.zeros((16,), jnp.float32)

    pltpu.sync_copy(r_hbm, r_v.at[pl.ds(0, _N)])
    pltpu.sync_copy(lab_hbm, lab_v.at[pl.ds(0, _N)])
    pltpu.sync_copy(bat_hbm, bat_v.at[pl.ds(0, _N)])

    def bs_step(k, _):
        sl = pl.ds(k * 16, 16)
        bs_v[sl] = jnp.where(lab_v[sl] == 0, r_v[sl], _NEG)
        return 0
    lax.fori_loop(0, _NBLK, bs_step, 0)

    starts = [jnp.int32(0)]
    for g in range(1, _G):
        def bstep(_, lohi, g=g):
            lo, hi = lohi
            mid = (lo + hi) // 2
            pred = bat_v[pl.ds(mid, 16)][0] < g
            return (jnp.where(pred, mid + 1, lo), jnp.where(pred, hi, mid))
        lo, _hi = lax.fori_loop(0, 14, bstep, (jnp.int32(0), jnp.int32(_N)))
        starts.append(lo)
    starts.append(jnp.int32(_N))

    base_i = wid * _CH

    bnd = jnp.where(iot == _G, _N, 0).astype(jnp.int32)
    for g in range(1, _G):
        bnd = bnd + jnp.where(iot == g, starts[g], 0)
    bnd_v[pl.ds(0, 16)] = bnd
    bnd_v[pl.ds(16, 16)] = jnp.zeros((16,), jnp.int32)

    def g_step(g, _):
        st = bnd_v[pl.ds(g, 16)][0]
        en = bnd_v[pl.ds(g + 1, 16)][0]
        a0 = st // 16
        b0 = en // 16
        jvA = a0 * 16 + iot
        vA = bs_v[pl.ds(a0 * 16, 16)]
        vA = jnp.where(jvA >= st, vA, _NEG)
        vA = jnp.where(jvA < en, vA, _NEG)
        jvB = b0 * 16 + iot
        vB = bs_v[pl.ds(b0 * 16, 16)]
        vB = jnp.where(jvB >= (a0 + 1) * 16, vB, _NEG)
        vB = jnp.where(jvB < en, vB, _NEG)

        rlo = jnp.maximum(base_i, st)
        rhi = jnp.minimum(base_i + _CH, en)
        nrb = jnp.maximum(0, (rhi - rlo + 15) // 16)

        def rb_step(rb, c):
            nacc, gacc = c
            rbase = rlo + rb * 16
            rowjv = rbase + iot
            av = r_v[pl.ds(rbase, 16)]
            lv = lab_v[pl.ds(rbase, 16)]
            c0 = _MARGIN - av
            c0 = jnp.where(lv > 0, c0, _NEG)
            c0 = jnp.where(rowjv < rhi, c0, _NEG)
            c0b = [c0[k] + zero16 for k in range(16)]

            nin = jnp.maximum(0, b0 - (a0 + 1))
            jbase0 = (a0 + 1) * 16

            def jstep2(k, accs):
                b1 = bs_v[pl.ds(jbase0 + k * 32, 16)]
                b2 = bs_v[pl.ds(jbase0 + k * 32 + 16, 16)]
                accs = tuple(accs[k2] + jnp.maximum(b1 + c0b[k2], 0.0)
                             for k2 in range(16))
                return tuple(accs[k2] + jnp.maximum(b2 + c0b[k2], 0.0)
                             for k2 in range(16))

            def jtail(t, accs):
                b = bs_v[pl.ds(jbase0 + (nin - 1) * 16, 16)]
                return tuple(accs[k2] + jnp.maximum(b + c0b[k2], 0.0)
                             for k2 in range(16))

            accs = lax.fori_loop(0, nin // 2, jstep2, (zero16,) * 16)
            accs = lax.fori_loop(0, nin % 2, jtail, accs)
            for ev in (vA, vB):
                accs = tuple(accs[k2] + jnp.maximum(ev + c0b[k2], 0.0)
                             for k2 in range(16))
            nacc = nacc + _tree_sum(list(accs))
            rmask = jnp.where(rowjv < rhi, jnp.float32(1.0), jnp.float32(0.0))
            gacc = gacc + jnp.where(lv > 0, rmask, 0.0)
            return (nacc, gacc)

        nv, gv = lax.fori_loop(0, nrb, rb_step, (zero16, zero16))
        out_v[pl.ds(g * 16, 16)] = nv
        out_v[pl.ds(128 + g * 16, 16)] = gv
        return 0

    lax.fori_loop(0, _G, g_step, 0)

    pltpu.sync_copy(out_v, part_hbm.at[wid])


_sc_kernel = functools.partial(
    pl.kernel,
    out_type=jax.ShapeDtypeStruct((_NW, 256), jnp.float32),
    mesh=plsc.VectorSubcoreMesh(
        core_axis_name="c", subcore_axis_name="s",
        num_cores=_NC, num_subcores=_NS),
    scratch_types=[
        pltpu.VMEM((_NPAD,), jnp.float32),
        pltpu.VMEM((_NPAD,), jnp.float32),
        pltpu.VMEM((_NPAD,), jnp.int32),
        pltpu.VMEM((_NPAD,), jnp.int32),
        pltpu.VMEM((32,), jnp.int32),
        pltpu.VMEM((256,), jnp.float32),
    ],
)(_sc_body)


def _combine_body(part_ref, bat_ref, r_ref, o_ref, out_ref):
    p = part_ref[...]
    b = bat_ref[...]
    d = r_ref[...] - o_ref[...]
    asum = jnp.sum(d * d)
    rsum = jnp.sum(jnp.abs(d))
    rank_num = jnp.float32(0.0)
    rank_den = jnp.float32(0.0)
    for g in range(_G):
        num_g = jnp.sum(p[:, g * 16:(g + 1) * 16])
        gold_g = jnp.sum(p[:, 128 + g * 16:144 + g * 16])
        size_g = jnp.sum((b == g).astype(jnp.float32))
        cnt = gold_g * (size_g - gold_g)
        lg = jnp.where(cnt > 0, num_g / jnp.maximum(cnt, 1.0), 0.0)
        rank_num = rank_num + lg
        rank_den = rank_den + jnp.where(cnt > 0, 1.0, 0.0)
    rank = jnp.where(rank_den > 0, rank_num / jnp.maximum(rank_den, 1.0), 0.0)
    align = asum / _N
    reg = rsum / _N
    total = rank + 0.5 * align + 0.1 * reg
    ri = lax.broadcasted_iota(jnp.int32, (8, 128), 0)
    li = lax.broadcasted_iota(jnp.int32, (8, 128), 1)
    vals = (jnp.where(li == 0, total, 0.0) + jnp.where(li == 1, rank, 0.0)
            + jnp.where(li == 2, align, 0.0) + jnp.where(li == 3, reg, 0.0))
    out_ref[...] = jnp.where(ri == 0, vals, 0.0)


def _combine(part, bat_pad, r_pad, o_pad):
    return pl.pallas_call(
        _combine_body,
        out_shape=jax.ShapeDtypeStruct((8, 128), jnp.float32),
    )(part, bat_pad, r_pad, o_pad)


def kernel(refined_scores, original_scores, node_labels, batch):
    r = refined_scores.astype(jnp.float32)
    o = original_scores.astype(jnp.float32)
    lab = node_labels.astype(jnp.int32)
    bat = batch.astype(jnp.int32)
    part = _sc_kernel(r, lab, bat)
    pad = _NW * _CH - _N
    bat_pad = jnp.pad(bat, (0, pad), constant_values=_G).reshape(80, 128)
    r_pad = jnp.pad(r, (0, pad)).reshape(80, 128)
    o_pad = jnp.pad(o, (0, pad)).reshape(80, 128)
    res = _combine(part, bat_pad, r_pad, o_pad)
    return (res[0, 0], res[0, 1], res[0, 2], res[0, 3])


# speedup vs baseline: 1.2070x; 1.1359x over previous
import functools

import jax
import jax.numpy as jnp
from jax import lax
from jax.experimental import pallas as pl
from jax.experimental.pallas import tpu as pltpu
from jax.experimental.pallas import tpu_sc as plsc

_N = 10000
_G = 8
_NC = 2
_NS = 16
_NW = _NC * _NS
_CH = 320
_NBLK = _N // 16
_NPAD = _N + 16
_CSC = _CH + 16 * (_G + 1) + 16
_MARGIN = 0.1
_NEG = -1e30


def _tree_sum(vs):
    while len(vs) > 1:
        vs = [vs[i] + vs[i + 1] for i in range(0, len(vs) - 1, 2)] + (
            [vs[-1]] if len(vs) % 2 else [])
    return vs[0]


def _sc_body(r_hbm, lab_hbm, bat_hbm, part_hbm,
             r_v, bs_v, lab_v, bat_v, bnd_v, csc_v, rst_v, out_v):
    wid = lax.axis_index("s") * _NC + lax.axis_index("c")
    iot = lax.iota(jnp.int32, 16)
    zero16 = jnp.zeros((16,), jnp.float32)

    pltpu.sync_copy(r_hbm, r_v.at[pl.ds(0, _N)])
    pltpu.sync_copy(lab_hbm, lab_v.at[pl.ds(0, _N)])
    pltpu.sync_copy(bat_hbm, bat_v.at[pl.ds(0, _N)])

    def bs_step(k, _):
        sl = pl.ds(k * 16, 16)
        bs_v[sl] = jnp.where(lab_v[sl] == 0, r_v[sl], _NEG)
        return 0
    lax.fori_loop(0, _NBLK, bs_step, 0)

    starts = [jnp.int32(0)]
    for g in range(1, _G):
        def bstep(_, lohi, g=g):
            lo, hi = lohi
            mid = (lo + hi) // 2
            pred = bat_v[pl.ds(mid, 16)][0] < g
            return (jnp.where(pred, mid + 1, lo), jnp.where(pred, hi, mid))
        lo, _hi = lax.fori_loop(0, 14, bstep, (jnp.int32(0), jnp.int32(_N)))
        starts.append(lo)
    starts.append(jnp.int32(_N))

    base_i = wid * _CH

    bnd = jnp.where(iot == _G, _N, 0).astype(jnp.int32)
    for g in range(1, _G):
        bnd = bnd + jnp.where(iot == g, starts[g], 0)
    bnd_v[pl.ds(0, 16)] = bnd
    bnd_v[pl.ds(16, 16)] = jnp.zeros((16,), jnp.int32)

    def gc_step(g, carry):
        off, rstart_vec, cnt_vec = carry
        st = bnd_v[pl.ds(g, 16)][0]
        en = bnd_v[pl.ds(g + 1, 16)][0]
        rlo = jnp.maximum(base_i, st)
        rhi = jnp.minimum(base_i + _CH, en)
        nfb = jnp.maximum(0, (rhi - rlo + 15) // 16)

        def comp_step(fb, off2):
            rbase = rlo + fb * 16
            rowjv = rbase + iot
            lv = lab_v[pl.ds(rbase, 16)]
            lv = jnp.where(rowjv < rhi, lv, 0)
            s = jnp.where(lv > 0, 1, 0)

            def _vgather(vec, idx):
                return lax.gather(
                    vec, idx[:, None],
                    lax.GatherDimensionNumbers(
                        offset_dims=(), collapsed_slice_dims=(0,),
                        start_index_map=(0,)),
                    (1,), mode=lax.GatherScatterMode.PROMISE_IN_BOUNDS)

            p = s
            for k in (1, 2, 4, 8):
                sh = _vgather(p, jnp.maximum(iot - k, 0))
                p = p + jnp.where(iot >= k, sh, 0)
            lo2 = jnp.zeros((16,), jnp.int32)
            for step in (8, 4, 2, 1):
                mid = lo2 + step
                pm = _vgather(p, mid - 1)
                lo2 = jnp.where(pm <= iot, mid, lo2)
            c0 = _MARGIN - r_v[pl.ds(rbase, 16)]
            csc_v[pl.ds(off2, 16)] = _vgather(c0, lo2)
            return off2 + p[15]

        off_end = lax.fori_loop(0, nfb, comp_step, off)
        rstart_vec = jnp.where(iot == g, off, rstart_vec)
        cnt_vec = jnp.where(iot == g, off_end - off, cnt_vec)
        csc_v[pl.ds(off_end, 16)] = jnp.full((16,), _NEG, jnp.float32)
        return (off_end + 16, rstart_vec, cnt_vec)

    zero16i = jnp.zeros((16,), jnp.int32)
    _off, rstart_vec, cnt_vec = lax.fori_loop(
        0, _G, gc_step, (jnp.int32(0), zero16i, zero16i))
    rst_v[pl.ds(0, 16)] = rstart_vec
    rst_v[pl.ds(16, 16)] = cnt_vec
    rst_v[pl.ds(32, 16)] = zero16i
    out_v[pl.ds(128, 16)] = cnt_vec.astype(jnp.float32)

    def g_step(g, _):
        st = bnd_v[pl.ds(g, 16)][0]
        en = bnd_v[pl.ds(g + 1, 16)][0]
        a0 = st // 16
        b0 = en // 16
        jvA = a0 * 16 + iot
        vA = bs_v[pl.ds(a0 * 16, 16)]
        vA = jnp.where(jvA >= st, vA, _NEG)
        vA = jnp.where(jvA < en, vA, _NEG)
        jvB = b0 * 16 + iot
        vB = bs_v[pl.ds(b0 * 16, 16)]
        vB = jnp.where(jvB >= (a0 + 1) * 16, vB, _NEG)
        vB = jnp.where(jvB < en, vB, _NEG)

        rstart = rst_v[pl.ds(g, 16)][0]
        rcnt = rst_v[pl.ds(16 + g, 16)][0]
        nrb = (rcnt + 15) // 16

        def rb_step(rb, c):
            nacc, gacc = c
            c0 = csc_v[pl.ds(rstart + rb * 16, 16)]
            c0b = [c0[k] + zero16 for k in range(16)]

            nin = jnp.maximum(0, b0 - (a0 + 1))
            jbase0 = (a0 + 1) * 16

            def jstep2(k, accs):
                b1 = bs_v[pl.ds(jbase0 + k * 32, 16)]
                b2 = bs_v[pl.ds(jbase0 + k * 32 + 16, 16)]
                accs = tuple(accs[k2] + jnp.maximum(b1 + c0b[k2], 0.0)
                             for k2 in range(16))
                return tuple(accs[k2] + jnp.maximum(b2 + c0b[k2], 0.0)
                             for k2 in range(16))

            def jtail(t, accs):
                b = bs_v[pl.ds(jbase0 + (nin - 1) * 16, 16)]
                return tuple(accs[k2] + jnp.maximum(b + c0b[k2], 0.0)
                             for k2 in range(16))

            accs = lax.fori_loop(0, nin // 2, jstep2, (zero16,) * 16)
            accs = lax.fori_loop(0, nin % 2, jtail, accs)
            for ev in (vA, vB):
                accs = tuple(accs[k2] + jnp.maximum(ev + c0b[k2], 0.0)
                             for k2 in range(16))
            nacc = nacc + _tree_sum(list(accs))
            return (nacc, gacc)

        nv, _gv = lax.fori_loop(0, nrb, rb_step, (zero16, zero16))
        out_v[pl.ds(g * 16, 16)] = nv
        return 0

    lax.fori_loop(0, _G, g_step, 0)

    pltpu.sync_copy(out_v, part_hbm.at[wid])


_sc_kernel = functools.partial(
    pl.kernel,
    out_type=jax.ShapeDtypeStruct((_NW, 144), jnp.float32),
    mesh=plsc.VectorSubcoreMesh(
        core_axis_name="c", subcore_axis_name="s",
        num_cores=_NC, num_subcores=_NS),
    scratch_types=[
        pltpu.VMEM((_NPAD,), jnp.float32),
        pltpu.VMEM((_NPAD,), jnp.float32),
        pltpu.VMEM((_NPAD,), jnp.int32),
        pltpu.VMEM((_NPAD,), jnp.int32),
        pltpu.VMEM((32,), jnp.int32),
        pltpu.VMEM((_CSC,), jnp.float32),
        pltpu.VMEM((48,), jnp.int32),
        pltpu.VMEM((144,), jnp.float32),
    ],
)(_sc_body)


def _combine_body(part_ref, bat_ref, r_ref, o_ref, out_ref):
    p = part_ref[...]
    b = bat_ref[...]
    d = r_ref[...] - o_ref[...]
    asum = jnp.sum(d * d)
    rsum = jnp.sum(jnp.abs(d))
    rank_num = jnp.float32(0.0)
    rank_den = jnp.float32(0.0)
    for g in range(_G):
        num_g = jnp.sum(p[:, g * 16:(g + 1) * 16])
        gold_g = jnp.sum(p[:, 128 + g:129 + g])
        size_g = jnp.sum((b == g).astype(jnp.float32))
        cnt = gold_g * (size_g - gold_g)
        lg = jnp.where(cnt > 0, num_g / jnp.maximum(cnt, 1.0), 0.0)
        rank_num = rank_num + lg
        rank_den = rank_den + jnp.where(cnt > 0, 1.0, 0.0)
    rank = jnp.where(rank_den > 0, rank_num / jnp.maximum(rank_den, 1.0), 0.0)
    align = asum / _N
    reg = rsum / _N
    total = rank + 0.5 * align + 0.1 * reg
    ri = lax.broadcasted_iota(jnp.int32, (8, 128), 0)
    li = lax.broadcasted_iota(jnp.int32, (8, 128), 1)
    vals = (jnp.where(li == 0, total, 0.0) + jnp.where(li == 1, rank, 0.0)
            + jnp.where(li == 2, align, 0.0) + jnp.where(li == 3, reg, 0.0))
    out_ref[...] = jnp.where(ri == 0, vals, 0.0)


def _combine(part, bat_pad, r_pad, o_pad):
    return pl.pallas_call(
        _combine_body,
        out_shape=jax.ShapeDtypeStruct((8, 128), jnp.float32),
    )(part, bat_pad, r_pad, o_pad)


def kernel(refined_scores, original_scores, node_labels, batch):
    r = refined_scores.astype(jnp.float32)
    o = original_scores.astype(jnp.float32)
    lab = node_labels.astype(jnp.int32)
    bat = batch.astype(jnp.int32)
    part = _sc_kernel(r, lab, bat)
    pad = _NW * _CH - _N
    bat_pad = jnp.pad(bat, (0, pad), constant_values=_G).reshape(80, 128)
    r_pad = jnp.pad(r, (0, pad)).reshape(80, 128)
    o_pad = jnp.pad(o, (0, pad)).reshape(80, 128)
    res = _combine(part, bat_pad, r_pad, o_pad)
    return (res[0, 0], res[0, 1], res[0, 2], res[0, 3])


# speedup vs baseline: 1.2882x; 1.0673x over previous
import functools

import jax
import jax.numpy as jnp
from jax import lax
from jax.experimental import pallas as pl
from jax.experimental.pallas import tpu as pltpu
from jax.experimental.pallas import tpu_sc as plsc

_N = 10000
_G = 8
_NC = 2
_NS = 16
_NW = _NC * _NS
_CH = 320
_NBLK = _N // 16
_NPAD = _N + 16
_CSC = _CH + 16 * (_G + 1) + 16
_MARGIN = 0.1
_NEG = -1e30


def _vgather(vec, idx):
    return lax.gather(
        vec, idx[:, None],
        lax.GatherDimensionNumbers(
            offset_dims=(), collapsed_slice_dims=(0,), start_index_map=(0,)),
        (1,), mode=lax.GatherScatterMode.PROMISE_IN_BOUNDS)


def _prefix_and_inverse(s, iot):
    p = s
    for k in (1, 2, 4, 8):
        sh = _vgather(p, jnp.maximum(iot - k, 0))
        p = p + jnp.where(iot >= k, sh, 0)
    lo2 = jnp.zeros((16,), jnp.int32)
    for step in (8, 4, 2, 1):
        mid = lo2 + step
        pm = _vgather(p, mid - 1)
        lo2 = jnp.where(pm <= iot, mid, lo2)
    return p, lo2


def _tree_sum(vs):
    while len(vs) > 1:
        vs = [vs[i] + vs[i + 1] for i in range(0, len(vs) - 1, 2)] + (
            [vs[-1]] if len(vs) % 2 else [])
    return vs[0]


def _sc_body(r_hbm, lab_hbm, bat_hbm, part_hbm,
             r_v, lab_v, bat_v, bnd_v, csc_v, rst_v, jsc_v, jst_v, out_v):
    wid = lax.axis_index("s") * _NC + lax.axis_index("c")
    iot = lax.iota(jnp.int32, 16)
    zero16 = jnp.zeros((16,), jnp.float32)

    pltpu.sync_copy(r_hbm, r_v.at[pl.ds(0, _N)])
    pltpu.sync_copy(lab_hbm, lab_v.at[pl.ds(0, _N)])
    pltpu.sync_copy(bat_hbm, bat_v.at[pl.ds(0, _N)])

    starts = [jnp.int32(0)]
    for g in range(1, _G):
        def bstep(_, lohi, g=g):
            lo, hi = lohi
            mid = (lo + hi) // 2
            pred = bat_v[pl.ds(mid, 16)][0] < g
            return (jnp.where(pred, mid + 1, lo), jnp.where(pred, hi, mid))
        lo, _hi = lax.fori_loop(0, 14, bstep, (jnp.int32(0), jnp.int32(_N)))
        starts.append(lo)
    starts.append(jnp.int32(_N))

    base_i = wid * _CH

    bnd = jnp.where(iot == _G, _N, 0).astype(jnp.int32)
    for g in range(1, _G):
        bnd = bnd + jnp.where(iot == g, starts[g], 0)
    bnd_v[pl.ds(0, 16)] = bnd
    bnd_v[pl.ds(16, 16)] = jnp.zeros((16,), jnp.int32)

    def gc_step(g, carry):
        off, rstart_vec, cnt_vec = carry
        st = bnd_v[pl.ds(g, 16)][0]
        en = bnd_v[pl.ds(g + 1, 16)][0]
        rlo = jnp.maximum(base_i, st)
        rhi = jnp.minimum(base_i + _CH, en)
        nfb = jnp.maximum(0, (rhi - rlo + 15) // 16)

        def comp_step(fb, off2):
            rbase = rlo + fb * 16
            rowjv = rbase + iot
            lv = lab_v[pl.ds(rbase, 16)]
            lv = jnp.where(rowjv < rhi, lv, 0)
            s = jnp.where(lv > 0, 1, 0)
            p, lo2 = _prefix_and_inverse(s, iot)
            c0 = _MARGIN - r_v[pl.ds(rbase, 16)]
            csc_v[pl.ds(off2, 16)] = _vgather(c0, lo2)
            return off2 + p[15]

        off_end = lax.fori_loop(0, nfb, comp_step, off)
        rstart_vec = jnp.where(iot == g, off, rstart_vec)
        cnt_vec = jnp.where(iot == g, off_end - off, cnt_vec)
        csc_v[pl.ds(off_end, 16)] = jnp.full((16,), _NEG, jnp.float32)
        return (off_end + 16, rstart_vec, cnt_vec)

    zero16i = jnp.zeros((16,), jnp.int32)
    _off, rstart_vec, cnt_vec = lax.fori_loop(
        0, _G, gc_step, (jnp.int32(0), zero16i, zero16i))
    rst_v[pl.ds(0, 16)] = rstart_vec
    rst_v[pl.ds(16, 16)] = cnt_vec
    rst_v[pl.ds(32, 16)] = zero16i
    out_v[pl.ds(128, 16)] = cnt_vec.astype(jnp.float32)

    def jc_step(g, carry):
        joff, jst_vec, jcnt_vec = carry
        st = bnd_v[pl.ds(g, 16)][0]
        en = bnd_v[pl.ds(g + 1, 16)][0]
        rcnt = rst_v[pl.ds(16 + g, 16)][0]
        a0 = st // 16
        nb = jnp.where(rcnt > 0, (en + 15) // 16 - a0, 0)

        def jcomp_step(bi, off2):
            base = (a0 + bi) * 16
            jv = base + iot
            lv = lab_v[pl.ds(base, 16)]
            lv = jnp.where(jv >= st, lv, 1)
            lv = jnp.where(jv < en, lv, 1)
            s = jnp.where(lv == 0, 1, 0)
            p, lo2 = _prefix_and_inverse(s, iot)
            vals = r_v[pl.ds(base, 16)]
            jsc_v[pl.ds(off2, 16)] = _vgather(vals, lo2)
            return off2 + p[15]

        joff_end = lax.fori_loop(0, nb, jcomp_step, joff)
        jst_vec = jnp.where(iot == g, joff, jst_vec)
        jcnt_vec = jnp.where(iot == g, joff_end - joff, jcnt_vec)
        jsc_v[pl.ds(joff_end, 16)] = jnp.full((16,), _NEG, jnp.float32)
        return (joff_end + 16, jst_vec, jcnt_vec)

    _joff, jst_vec, jcnt_vec = lax.fori_loop(
        0, _G, jc_step, (jnp.int32(0), zero16i, zero16i))
    jst_v[pl.ds(0, 16)] = jst_vec
    jst_v[pl.ds(16, 16)] = jcnt_vec
    jst_v[pl.ds(32, 16)] = zero16i

    def g_step(g, _):
        rstart = rst_v[pl.ds(g, 16)][0]
        rcnt = rst_v[pl.ds(16 + g, 16)][0]
        jstart = jst_v[pl.ds(g, 16)][0]
        jcnt = jst_v[pl.ds(16 + g, 16)][0]
        nrb = (rcnt + 15) // 16
        njb = (jcnt + 15) // 16

        def rb_step(rb, nacc):
            c0 = csc_v[pl.ds(rstart + rb * 16, 16)]
            c0b = [c0[k] + zero16 for k in range(16)]

            def jstep2(k, accs):
                b1 = jsc_v[pl.ds(jstart + k * 32, 16)]
                b2 = jsc_v[pl.ds(jstart + k * 32 + 16, 16)]
                accs = tuple(accs[k2] + jnp.maximum(b1 + c0b[k2], 0.0)
                             for k2 in range(16))
                return tuple(accs[k2] + jnp.maximum(b2 + c0b[k2], 0.0)
                             for k2 in range(16))

            def jtail(t, accs):
                b = jsc_v[pl.ds(jstart + (njb - 1) * 16, 16)]
                return tuple(accs[k2] + jnp.maximum(b + c0b[k2], 0.0)
                             for k2 in range(16))

            accs = lax.fori_loop(0, njb // 2, jstep2, (zero16,) * 16)
            accs = lax.fori_loop(0, njb % 2, jtail, accs)
            return nacc + _tree_sum(list(accs))

        nv = lax.fori_loop(0, nrb, rb_step, zero16)
        out_v[pl.ds(g * 16, 16)] = nv
        return 0

    lax.fori_loop(0, _G, g_step, 0)

    pltpu.sync_copy(out_v, part_hbm.at[wid])


_sc_kernel = functools.partial(
    pl.kernel,
    out_type=jax.ShapeDtypeStruct((_NW, 144), jnp.float32),
    mesh=plsc.VectorSubcoreMesh(
        core_axis_name="c", subcore_axis_name="s",
        num_cores=_NC, num_subcores=_NS),
    scratch_types=[
        pltpu.VMEM((_NPAD,), jnp.float32),
        pltpu.VMEM((_NPAD,), jnp.int32),
        pltpu.VMEM((_NPAD,), jnp.int32),
        pltpu.VMEM((32,), jnp.int32),
        pltpu.VMEM((_CSC,), jnp.float32),
        pltpu.VMEM((48,), jnp.int32),
        pltpu.VMEM((_N + 16 * (_G + 1) + 16,), jnp.float32),
        pltpu.VMEM((48,), jnp.int32),
        pltpu.VMEM((144,), jnp.float32),
    ],
)(_sc_body)


def _combine_body(part_ref, bat_ref, r_ref, o_ref, out_ref):
    p = part_ref[...]
    b = bat_ref[...]
    d = r_ref[...] - o_ref[...]
    asum = jnp.sum(d * d)
    rsum = jnp.sum(jnp.abs(d))
    rank_num = jnp.float32(0.0)
    rank_den = jnp.float32(0.0)
    for g in range(_G):
        num_g = jnp.sum(p[:, g * 16:(g + 1) * 16])
        gold_g = jnp.sum(p[:, 128 + g:129 + g])
        size_g = jnp.sum((b == g).astype(jnp.float32))
        cnt = gold_g * (size_g - gold_g)
        lg = jnp.where(cnt > 0, num_g / jnp.maximum(cnt, 1.0), 0.0)
        rank_num = rank_num + lg
        rank_den = rank_den + jnp.where(cnt > 0, 1.0, 0.0)
    rank = jnp.where(rank_den > 0, rank_num / jnp.maximum(rank_den, 1.0), 0.0)
    align = asum / _N
    reg = rsum / _N
    total = rank + 0.5 * align + 0.1 * reg
    ri = lax.broadcasted_iota(jnp.int32, (8, 128), 0)
    li = lax.broadcasted_iota(jnp.int32, (8, 128), 1)
    vals = (jnp.where(li == 0, total, 0.0) + jnp.where(li == 1, rank, 0.0)
            + jnp.where(li == 2, align, 0.0) + jnp.where(li == 3, reg, 0.0))
    out_ref[...] = jnp.where(ri == 0, vals, 0.0)


def _combine(part, bat_pad, r_pad, o_pad):
    return pl.pallas_call(
        _combine_body,
        out_shape=jax.ShapeDtypeStruct((8, 128), jnp.float32),
    )(part, bat_pad, r_pad, o_pad)


def kernel(refined_scores, original_scores, node_labels, batch):
    r = refined_scores.astype(jnp.float32)
    o = original_scores.astype(jnp.float32)
    lab = node_labels.astype(jnp.int32)
    bat = batch.astype(jnp.int32)
    part = _sc_kernel(r, lab, bat)
    pad = _NW * _CH - _N
    bat_pad = jnp.pad(bat, (0, pad), constant_values=_G).reshape(80, 128)
    r_pad = jnp.pad(r, (0, pad)).reshape(80, 128)
    o_pad = jnp.pad(o, (0, pad)).reshape(80, 128)
    res = _combine(part, bat_pad, r_pad, o_pad)
    return (res[0, 0], res[0, 1], res[0, 2], res[0, 3])
